# baseline (speedup 1.0000x reference)
import functools

import jax
import jax.numpy as jnp
from jax import lax
from jax.experimental import pallas as pl
from jax.experimental.pallas import tpu as pltpu
from jax.experimental.pallas import tpu_sc as plsc

_B = 4096
_L = 200
_D = 50
_DP = 56
_PITCH = 56
_NCLS = 1000
_CHUNK = 128
_NW = 32
_SPW = _B // _NW
_NIR = _SPW * _L // _CHUNK
_NG = 2 * _NIR


def _gather_sum_sc(idx2, table):
    mesh = plsc.VectorSubcoreMesh(core_axis_name="c", subcore_axis_name="s")

    @functools.partial(
        pl.kernel,
        mesh=mesh,
        compiler_params=pltpu.CompilerParams(use_tc_tiling_on_sc=False,
                                             needs_layout_passes=False),
        out_type=jax.ShapeDtypeStruct((_B, _DP), jnp.float32),
        scratch_types=[
            pltpu.VMEM((_NIR, _CHUNK), jnp.int32),
            pltpu.VMEM((_NG, _CHUNK), jnp.int32),
            pltpu.VMEM((_NG, 64), jnp.int32),
            pltpu.VMEM((_CHUNK, _D), jnp.float32),
            pltpu.VMEM((_CHUNK, _D), jnp.float32),
            pltpu.VMEM((_SPW, _DP), jnp.float32),
            pltpu.SemaphoreType.DMA,
            pltpu.SemaphoreType.DMA,
        ],
    )
    def k(idx_hbm, table_hbm, out_hbm, idx_v, gidx, shifts, buf_a, buf_b,
          out_v, sem_a, sem_b):
        cid = lax.axis_index("c")
        sid = lax.axis_index("s")
        w = sid * 2 + cid
        pltpu.sync_copy(idx_hbm.at[pl.ds(w * _NIR, _NIR)], idx_v)

        zero = jnp.zeros((16,), jnp.float32)
        iota = lax.iota(jnp.int32, 16)

        def zero_body(i, carry):
            for cb in (0, 16, 32, 40):
                out_v[i, pl.ds(cb, 16)] = zero
            return carry

        lax.fori_loop(0, _SPW, zero_body, 0)

        def tf_body(r, carry):
            for t in range(8):
                i = idx_v[r, pl.ds(16 * t, 16)]
                k0 = (_PITCH * i) // _D
                s = _PITCH * i - _D * k0
                g = 2 * r + (t // 4)
                grow = jnp.full((16,), 0, jnp.int32) + g
                mcol = 32 * (t % 4) + 2 * iota
                plsc.store_scatter(gidx, [grow, mcol], k0)
                plsc.store_scatter(gidx, [grow, mcol + 1], k0 + 1)
                scol = 16 * (t % 4) + iota
                plsc.store_scatter(shifts, [grow, scol], s)
            return carry

        lax.fori_loop(0, _NIR, tf_body, 0)

        def issue(g, buf, sem):
            return pltpu.async_copy(table_hbm.at[gidx.at[g]], buf, sem)

        def wait(buf, sem):
            pltpu.make_async_copy(table_hbm.at[gidx.at[0]], buf, sem).wait()

        lo2 = iota < 2

        def accumulate(g, buf, carry):
            def row_body(m, sc):
                smp, cnt = sc
                sft = plsc.load_gather(shifts, [iota * 0 + g, iota * 0 + m])
                for cb in (0, 16, 32, 34):
                    a = 100 * m + sft + (iota + cb)
                    row = a // _PITCH
                    col = a - _PITCH * row
                    x = plsc.load_gather(buf, [row, col])
                    if cb == 32:
                        x = jnp.where(lo2, x, 0.0)
                    plsc.addupdate(out_v.at[smp, pl.ds(cb, 16)], x)
                wrap = cnt == _L - 1
                smp = jnp.where(wrap, smp + 1, smp)
                cnt = jnp.where(wrap, 0, cnt + 1)
                return (smp, cnt)

            return lax.fori_loop(0, 64, row_body, carry, unroll=2)

        issue(0, buf_a, sem_a)

        def pair_body(u, carry):
            issue(2 * u + 1, buf_b, sem_b)
            wait(buf_a, sem_a)
            carry = accumulate(2 * u, buf_a, carry)

            @pl.when(u < _NG // 2 - 1)
            def _():
                issue(2 * u + 2, buf_a, sem_a)

            wait(buf_b, sem_b)
            return accumulate(2 * u + 1, buf_b, carry)

        lax.fori_loop(0, _NG // 2, pair_body, (jnp.int32(0), jnp.int32(0)))
        pltpu.sync_copy(out_v, out_hbm.at[pl.ds(w * _SPW, _SPW)])

    return k(idx2, table)


def _mlp_tc(sums, lens, w1t, b1r, w2t, b2r):
    bm = 512

    def body(s_ref, l_ref, w1_ref, b1_ref, w2_ref, b2_ref, o_ref):
        avg = s_ref[...] / l_ref[...]
        h = jnp.dot(avg, w1_ref[...], preferred_element_type=jnp.float32,
                    precision=lax.Precision.HIGHEST)
        h = jnp.maximum(h + b1_ref[...], 0.0)
        o_ref[...] = jnp.dot(h, w2_ref[...], preferred_element_type=jnp.float32,
                             precision=lax.Precision.HIGHEST) + b2_ref[...]

    return pl.pallas_call(
        body,
        grid=(_B // bm,),
        in_specs=[
            pl.BlockSpec((bm, _DP), lambda i: (i, 0)),
            pl.BlockSpec((bm, 1), lambda i: (i, 0)),
            pl.BlockSpec((_DP, _D), lambda i: (0, 0)),
            pl.BlockSpec((1, _D), lambda i: (0, 0)),
            pl.BlockSpec((_D, _NCLS), lambda i: (0, 0)),
            pl.BlockSpec((1, _NCLS), lambda i: (0, 0)),
        ],
        out_specs=pl.BlockSpec((bm, _NCLS), lambda i: (i, 0)),
        out_shape=jax.ShapeDtypeStruct((_B, _NCLS), jnp.float32),
    )(sums, lens, w1t, b1r, w2t, b2r)


def kernel(input_text, text_len, emb_table, W1, b1, W2, b2):
    idx2 = input_text.reshape(_B * _L // _CHUNK, _CHUNK).astype(jnp.int32)
    sums = _gather_sum_sc(idx2, emb_table)
    lens = text_len.astype(jnp.float32).reshape(_B, 1)
    w1t = jnp.pad(W1.T, ((0, _DP - _D), (0, 0)))
    return _mlp_tc(sums, lens, w1t, b1.reshape(1, _D), W2.T, b2.reshape(1, _NCLS))


# speedup vs baseline: 3.3832x; 3.3832x over previous
import functools

import jax
import jax.numpy as jnp
from jax import lax
from jax.experimental import pallas as pl
from jax.experimental.pallas import tpu as pltpu
from jax.experimental.pallas import tpu_sc as plsc

_B = 4096
_L = 200
_D = 50
_DP = 56
_NCLS = 1000
_CHUNK = 128
_NW = 32
_SPW = _B // _NW
_NCH = _SPW * _L // _CHUNK
_COLS = (0, 16, 32, 40)


def _gather_sum_sc(idx2, table56):
    mesh = plsc.VectorSubcoreMesh(core_axis_name="c", subcore_axis_name="s")

    @functools.partial(
        pl.kernel,
        mesh=mesh,
        compiler_params=pltpu.CompilerParams(use_tc_tiling_on_sc=False),
        out_type=jax.ShapeDtypeStruct((_B, _DP), jnp.float32),
        scratch_types=[
            pltpu.VMEM((_NCH, _CHUNK), jnp.int32),
            pltpu.VMEM((_CHUNK, _DP), jnp.float32),
            pltpu.VMEM((_CHUNK, _DP), jnp.float32),
            pltpu.VMEM((_SPW, _DP), jnp.float32),
            pltpu.SemaphoreType.DMA,
            pltpu.SemaphoreType.DMA,
        ],
    )
    def k(idx_hbm, table_hbm, out_hbm, idx_v, buf_a, buf_b, out_v, sem_a, sem_b):
        cid = lax.axis_index("c")
        sid = lax.axis_index("s")
        w = sid * 2 + cid
        pltpu.sync_copy(idx_hbm.at[pl.ds(w * _NCH, _NCH)], idx_v)

        zero = jnp.zeros((16,), jnp.float32)

        def zero_body(i, carry):
            for cb in _COLS:
                out_v[i, pl.ds(cb, 16)] = zero
            return carry

        lax.fori_loop(0, _SPW, zero_body, 0)

        def issue(c, buf, sem):
            return pltpu.async_copy(table_hbm.at[idx_v.at[c]], buf, sem)

        def wait(buf, sem):
            pltpu.make_async_copy(table_hbm.at[idx_v.at[0]], buf, sem).wait()

        lo8 = lax.iota(jnp.int32, 16) < 8

        def accumulate(buf, carry):
            def row_body(j, sc):
                s, cnt = sc
                for cb in _COLS:
                    x = buf[j, pl.ds(cb, 16)]
                    if cb == 32:
                        x = jnp.where(lo8, x, 0.0)
                    plsc.addupdate(out_v.at[s, pl.ds(cb, 16)], x)
                wrap = cnt == _L - 1
                s = jnp.where(wrap, s + 1, s)
                cnt = jnp.where(wrap, 0, cnt + 1)
                return (s, cnt)

            return lax.fori_loop(0, _CHUNK, row_body, carry, unroll=4)

        issue(0, buf_a, sem_a)

        def pair_body(t, carry):
            issue(2 * t + 1, buf_b, sem_b)
            wait(buf_a, sem_a)
            carry = accumulate(buf_a, carry)

            @pl.when(t < _NCH // 2 - 1)
            def _():
                issue(2 * t + 2, buf_a, sem_a)

            wait(buf_b, sem_b)
            return accumulate(buf_b, carry)

        lax.fori_loop(0, _NCH // 2, pair_body, (jnp.int32(0), jnp.int32(0)))
        pltpu.sync_copy(out_v, out_hbm.at[pl.ds(w * _SPW, _SPW)])

    return k(idx2, table56)


def _pad_tc(table):
    rb = 25000
    v = table.shape[0]

    def body(t_ref, o_ref):
        o_ref[:, pl.ds(0, _D)] = t_ref[...]
        o_ref[:, pl.ds(_D, _DP - _D)] = jnp.zeros((rb, _DP - _D), jnp.float32)

    return pl.pallas_call(
        body,
        grid=(v // rb,),
        in_specs=[pl.BlockSpec((rb, _D), lambda i: (i, 0))],
        out_specs=pl.BlockSpec((rb, _DP), lambda i: (i, 0)),
        out_shape=jax.ShapeDtypeStruct((v, _DP), jnp.float32),
    )(table)


def _mlp_tc(sums, lens, w1t, b1r, w2t, b2r):
    bm = 512

    def body(s_ref, l_ref, w1_ref, b1_ref, w2_ref, b2_ref, o_ref):
        avg = s_ref[...] / l_ref[...]
        h = jnp.dot(avg, w1_ref[...], preferred_element_type=jnp.float32,
                    precision=lax.Precision.HIGHEST)
        h = jnp.maximum(h + b1_ref[...], 0.0)
        o_ref[...] = jnp.dot(h, w2_ref[...], preferred_element_type=jnp.float32,
                             precision=lax.Precision.HIGHEST) + b2_ref[...]

    return pl.pallas_call(
        body,
        grid=(_B // bm,),
        in_specs=[
            pl.BlockSpec((bm, _DP), lambda i: (i, 0)),
            pl.BlockSpec((bm, 1), lambda i: (i, 0)),
            pl.BlockSpec((_DP, _D), lambda i: (0, 0)),
            pl.BlockSpec((1, _D), lambda i: (0, 0)),
            pl.BlockSpec((_D, _NCLS), lambda i: (0, 0)),
            pl.BlockSpec((1, _NCLS), lambda i: (0, 0)),
        ],
        out_specs=pl.BlockSpec((bm, _NCLS), lambda i: (i, 0)),
        out_shape=jax.ShapeDtypeStruct((_B, _NCLS), jnp.float32),
    )(sums, lens, w1t, b1r, w2t, b2r)


def kernel(input_text, text_len, emb_table, W1, b1, W2, b2):
    idx2 = input_text.reshape(_B * _L // _CHUNK, _CHUNK).astype(jnp.int32)
    table56 = _pad_tc(emb_table)
    sums = _gather_sum_sc(idx2, table56)
    lens = text_len.astype(jnp.float32).reshape(_B, 1)
    w1t = jnp.pad(W1.T, ((0, _DP - _D), (0, 0)))
    return _mlp_tc(sums, lens, w1t, b1.reshape(1, _D), W2.T, b2.reshape(1, _NCLS))
